# baseline (speedup 1.0000x reference)
import jax
import jax.numpy as jnp
from jax.experimental import pallas as pl
from jax.experimental.pallas import tpu as pltpu

_N = 4096
_D = 128
_K = 2
_BI = 256
_NBUF = 4
_NSTEP = _N // _BI


def _fused_kernel(h_ref, w_ref, adj_ref, out_ref, hw_ref, buf_ref, sem_ref):
    def copy(step, slot, k):
        return pltpu.make_async_copy(
            adj_ref.at[k, pl.ds(step * _BI, _BI), :],
            buf_ref.at[slot, k],
            sem_ref.at[slot, k],
        )

    for b in range(_NBUF):
        copy(b, b, 0).start()
        copy(b, b, 1).start()

    hw_ref[...] = jnp.dot(h_ref[...], w_ref[...],
                          preferred_element_type=jnp.float32)

    def body(step, carry):
        slot = jax.lax.rem(step, _NBUF)
        copy(step, slot, 0).wait()
        copy(step, slot, 1).wait()
        a = buf_ref[slot, 0] + buf_ref[slot, 1]
        out_ref[pl.ds(step * _BI, _BI), :] = jnp.dot(
            a, hw_ref[...], preferred_element_type=jnp.float32)

        @pl.when(step + _NBUF < _NSTEP)
        def _():
            copy(step + _NBUF, slot, 0).start()
            copy(step + _NBUF, slot, 1).start()

        return carry

    jax.lax.fori_loop(0, _NSTEP, body, 0)


def kernel(h, Adj, weight, bias):
    del bias
    return pl.pallas_call(
        _fused_kernel,
        in_specs=[
            pl.BlockSpec(memory_space=pltpu.MemorySpace.VMEM),
            pl.BlockSpec(memory_space=pltpu.MemorySpace.VMEM),
            pl.BlockSpec(memory_space=pltpu.MemorySpace.HBM),
        ],
        out_specs=pl.BlockSpec(memory_space=pltpu.MemorySpace.VMEM),
        out_shape=jax.ShapeDtypeStruct((_N, _D), jnp.float32),
        scratch_shapes=[
            pltpu.VMEM((_N, _D), jnp.float32),
            pltpu.VMEM((_NBUF, _K, _BI, _N), jnp.float32),
            pltpu.SemaphoreType.DMA((_NBUF, _K)),
        ],
    )(h, weight, Adj)
